# baseline (speedup 1.0000x reference)
import functools

import jax
import jax.numpy as jnp
from jax import lax
from jax.experimental import pallas as pl
from jax.experimental.pallas import tpu as pltpu
from jax.experimental.pallas import tpu_sc as plsc

_SPAN = 64
_B = 16
_S = 2048
_L = 16
_NV = _S // _L
_PAD = 8


def _bcast_last(x):
    idx = jnp.full((_L,), _L - 1, dtype=jnp.int32)
    return x.at[idx].get(mode="promise_in_bounds")


_mesh = plsc.VectorSubcoreMesh(core_axis_name="c", subcore_axis_name="s", num_cores=1)


@functools.partial(
    pl.kernel,
    mesh=_mesh,
    out_type=jax.ShapeDtypeStruct((_B, _S), jnp.int32),
    compiler_params=pltpu.CompilerParams(needs_layout_passes=False),
    scratch_types=[
        pltpu.VMEM((_S,), jnp.int32),
        pltpu.VMEM((_S,), jnp.int32),
    ],
)
def _seg_kernel(mask_hbm, out_hbm, m_buf, seg_buf):
    w = lax.axis_index("s")

    @pl.when(w < _B)
    def _():
        pltpu.sync_copy(mask_hbm.at[w], m_buf)
        lane = lax.iota(jnp.int32, _L)
        lane1_u = (lane + 1).astype(jnp.uint32)

        def step(i, cur, prv, boundary0, carries):
            rs_c, cid_c, lc_c = carries
            t1 = lane1_u + jnp.uint32(i * _L)
            change = cur != prv
            boundary = change | boundary0 if boundary0 is not None else change
            bidx = jnp.where(boundary, t1, jnp.uint32(0))
            rs = jnp.maximum(plsc.cummax(bidx), rs_c)
            cb = ((t1 - rs) & (_SPAN - 1)) == 0
            cid = plsc.cumsum(jnp.where(cb, jnp.int32(1), jnp.int32(0))) + cid_c
            lc = jnp.maximum(lc_c, bidx)
            seg_buf[pl.ds(i * _L, _L)] = cid
            return (_bcast_last(rs), _bcast_last(cid), lc)

        zeros_u = jnp.zeros((_L,), jnp.uint32)
        neg1 = jnp.full((_L,), -1, jnp.int32)
        cur0 = m_buf[pl.ds(0, _L)]
        prv0 = cur0.at[jnp.maximum(lane - 1, 0)].get(mode="promise_in_bounds")
        carries = step(0, cur0, prv0, lane == 0, (zeros_u, neg1, zeros_u))

        @plsc.parallel_loop(1, _NV, carry=carries, unroll=16)
        def loop_carry(i, carries):
            cur = m_buf[pl.ds(i * _L, _L)]
            prv = m_buf[pl.ds(i * _L - 1, _L)]
            return step(i, cur, prv, None, carries)

        _, _, lc_f = loop_carry

        last_vreg = m_buf[pl.ds(_S - _L, _L)]
        keep_last = jnp.max(jnp.where(lane == _L - 1, last_vreg, jnp.int32(0)))
        extent = jnp.where(keep_last == 1, jnp.int32(_S),
                           jnp.max(lc_f).astype(jnp.int32))

        @plsc.parallel_loop(extent // _L, _NV)
        def _fix(i):
            t = lane + i * _L
            v = seg_buf[pl.ds(i * _L, _L)]
            seg_buf[pl.ds(i * _L, _L)] = jnp.where(t < extent, v, jnp.int32(-1))

        pltpu.sync_copy(seg_buf, out_hbm.at[w])


def kernel(inp, padding_mask, regular_tokens_mask):
    del inp, padding_mask
    return _seg_kernel(regular_tokens_mask)


# speedup vs baseline: 1.0312x; 1.0312x over previous
import functools

import jax
import jax.numpy as jnp
from jax import lax
from jax.experimental import pallas as pl
from jax.experimental.pallas import tpu as pltpu
from jax.experimental.pallas import tpu_sc as plsc

_SPAN = 64
_B = 16
_S = 2048
_L = 16
_NV = _S // _L
_PAD = 8


def _bcast_last(x):
    idx = jnp.full((_L,), _L - 1, dtype=jnp.int32)
    return x.at[idx].get(mode="promise_in_bounds")


_mesh = plsc.VectorSubcoreMesh(core_axis_name="c", subcore_axis_name="s", num_cores=1)


@functools.partial(
    pl.kernel,
    mesh=_mesh,
    out_type=jax.ShapeDtypeStruct((_B, _S), jnp.int32),
    compiler_params=pltpu.CompilerParams(needs_layout_passes=False),
    scratch_types=[
        pltpu.VMEM((_S,), jnp.int32),
        pltpu.VMEM((_S,), jnp.int32),
    ],
)
def _seg_kernel(mask_hbm, out_hbm, m_buf, seg_buf):
    w = lax.axis_index("s")

    @pl.when(w < _B)
    def _():
        pltpu.sync_copy(mask_hbm.at[w], m_buf)
        lane = lax.iota(jnp.int32, _L)
        lane1_u = (lane + 1).astype(jnp.uint32)

        def step(i, cur, prv, boundary0, carries):
            rs_c, cid_c, lc_c = carries
            t1 = lane1_u + jnp.uint32(i * _L)
            change = cur != prv
            boundary = change | boundary0 if boundary0 is not None else change
            bidx = jnp.where(boundary, t1, jnp.uint32(0))
            rs = jnp.maximum(plsc.cummax(bidx), rs_c)
            cb = ((t1 - rs) & (_SPAN - 1)) == 0
            cid = plsc.cumsum(jnp.where(cb, jnp.int32(1), jnp.int32(0))) + cid_c
            lc = jnp.maximum(lc_c, bidx)
            seg_buf[pl.ds(i * _L, _L)] = cid
            return (_bcast_last(rs), _bcast_last(cid), lc)

        zeros_u = jnp.zeros((_L,), jnp.uint32)
        neg1 = jnp.full((_L,), -1, jnp.int32)
        cur0 = m_buf[pl.ds(0, _L)]
        prv0 = cur0.at[jnp.maximum(lane - 1, 0)].get(mode="promise_in_bounds")
        carries = step(0, cur0, prv0, lane == 0, (zeros_u, neg1, zeros_u))

        @plsc.parallel_loop(1, _NV, carry=carries, unroll=4)
        def loop_carry(i, carries):
            cur = m_buf[pl.ds(i * _L, _L)]
            prv = m_buf[pl.ds(i * _L - 1, _L)]
            return step(i, cur, prv, None, carries)

        _, _, lc_f = loop_carry

        last_vreg = m_buf[pl.ds(_S - _L, _L)]
        keep_last = jnp.max(jnp.where(lane == _L - 1, last_vreg, jnp.int32(0)))
        extent = jnp.where(keep_last == 1, jnp.int32(_S),
                           jnp.max(lc_f).astype(jnp.int32))

        @plsc.parallel_loop(extent // _L, _NV)
        def _fix(i):
            t = lane + i * _L
            v = seg_buf[pl.ds(i * _L, _L)]
            seg_buf[pl.ds(i * _L, _L)] = jnp.where(t < extent, v, jnp.int32(-1))

        pltpu.sync_copy(seg_buf, out_hbm.at[w])


def kernel(inp, padding_mask, regular_tokens_mask):
    del inp, padding_mask
    return _seg_kernel(regular_tokens_mask)
